# baseline (speedup 1.0000x reference)
import functools

import jax
import jax.numpy as jnp
from jax.experimental import pallas as pl


def _body(e_ref, qc_ref, cq_ref, out_ref):
    i = pl.program_id(0)
    msg = jnp.dot(qc_ref[...], e_ref[...], preferred_element_type=jnp.float32)
    contrib = jnp.dot(cq_ref[...], msg, preferred_element_type=jnp.float32)

    @pl.when(i == 0)
    def _init():
        out_ref[...] = contrib

    @pl.when(i > 0)
    def _acc():
        out_ref[...] += contrib


@functools.partial(jax.jit, static_argnames=())
def kernel(skill_embs, HG_qc, HG_cq):
    n_edges, n_skills = HG_qc.shape
    d = skill_embs.shape[1]
    BE = 128

    return pl.pallas_call(
        _body,
        grid=(n_edges // BE,),
        in_specs=[
            pl.BlockSpec((n_skills, d), lambda i: (0, 0)),
            pl.BlockSpec((BE, n_skills), lambda i: (i, 0)),
            pl.BlockSpec((n_skills, BE), lambda i: (0, i)),
        ],
        out_specs=pl.BlockSpec((n_skills, d), lambda i: (0, 0)),
        out_shape=jax.ShapeDtypeStruct((n_skills, d), jnp.float32),
    )(skill_embs, HG_qc, HG_cq)
